# Initial kernel scaffold; baseline (speedup 1.0000x reference)
import jax
import jax.numpy as jnp
from jax.experimental import pallas as pl


def kernel(x_nchw, skip_nchw, w_hwio, gamma, beta):
    raise NotImplementedError("write your pallas kernel here")


# speedup vs baseline: 1.0634x; 1.0634x over previous
import functools
import math

import jax
import jax.numpy as jnp
from jax.experimental import pallas as pl
from jax.experimental.pallas import tpu as pltpu

_EPS = 1e-5


def _round_up(v, m):
    return ((v + m - 1) // m) * m


def _width_matrix(n_in, n_out):
    if n_in == 1:
        return jnp.ones((1, n_out), jnp.float32)
    src = jnp.arange(n_out, dtype=jnp.float32) * (n_in - 1) / (n_out - 1)
    i0 = jnp.clip(jnp.floor(src).astype(jnp.int32), 0, n_in - 1)
    i1 = jnp.clip(i0 + 1, 0, n_in - 1)
    frac = src - i0.astype(jnp.float32)
    cols = jnp.arange(n_out)
    m = jnp.zeros((n_out, n_in), jnp.float32)
    m = m.at[cols, i0].add(1.0 - frac)
    m = m.at[cols, i1].add(frac)
    return m.T


def _height_taps(n_in, n_out):
    taps = []
    for dst in range(n_out):
        if n_in == 1:
            taps.append((0, 0, 1.0, 0.0))
            continue
        src = dst * (n_in - 1) / (n_out - 1)
        i0 = min(int(math.floor(src)), n_in - 1)
        i1 = min(i0 + 1, n_in - 1)
        frac = src - i0
        taps.append((i0, i1, 1.0 - frac, float(frac)))
    return tuple(taps)


def _fwd_kernel(aw_ref, ws_ref, x_ref, s_ref, y_ref, sum_ref, ssq_ref,
                t1_s, img_s, rs_s, *, h_taps, h2, w2, cin_p, cout):
    p = h2 * w2
    c2 = 2 * cin_p
    f32 = jnp.float32
    bf16 = jnp.bfloat16

    t1_s[...] = jnp.dot(x_ref[0], aw_ref[...], preferred_element_type=f32)

    for t in range(h2 // 2):
        halves = []
        for hh in (2 * t, 2 * t + 1):
            i0, i1, a0, a1 = h_taps[hh]
            r = a0 * t1_s[i0 * cin_p:(i0 + 1) * cin_p, :]
            if a1 != 0.0:
                r = r + a1 * t1_s[i1 * cin_p:(i1 + 1) * cin_p, :]
            halves.append(r)
        img_s[0:cin_p, 2 * t * w2:(2 * t + 2) * w2] = (
            jnp.concatenate(halves, axis=1).astype(bf16))

    img_s[cin_p:c2, :] = s_ref[0].astype(bf16)

    lin = jax.lax.broadcasted_iota(jnp.int32, (1, p), 1)
    wpos = lin - (lin // w2) * w2
    zero = jnp.zeros((), bf16)
    c_blk = img_s[0:c2, :]
    img_s[c2:2 * c2, :] = jnp.where(wpos >= 1,
                                    jnp.roll(c_blk, 1, axis=1), zero)
    img_s[2 * c2:3 * c2, :] = jnp.where(wpos <= w2 - 2,
                                        jnp.roll(c_blk, -1, axis=1), zero)

    rs_s[...] = jnp.dot(ws_ref[...], img_s[...],
                        preferred_element_type=f32).astype(bf16)

    mid = rs_s[cout:2 * cout, :].astype(f32)
    top = jnp.where(lin >= w2,
                    jnp.roll(rs_s[0:cout, :], w2, axis=1), zero).astype(f32)
    bot = jnp.where(lin < p - w2,
                    jnp.roll(rs_s[2 * cout:3 * cout, :], -w2, axis=1),
                    zero).astype(f32)
    y = jnp.maximum(mid + top + bot, 0.0)

    sum_ref[0] = jnp.sum(y, axis=1, keepdims=True)
    ssq_ref[0] = jnp.sum(y * y, axis=1, keepdims=True)
    y_ref[0] = y.astype(bf16)


def _bn_kernel(scale_ref, shift_ref, y_ref, o_ref):
    o_ref[0] = (y_ref[0].astype(jnp.float32) * scale_ref[...]
                + shift_ref[...])


def kernel(x_nchw, skip_nchw, w_hwio, gamma, beta):
    n, cin, h, w = x_nchw.shape
    _, cin_s, h2, w2 = skip_nchw.shape
    kh, kw, cin2, cout = w_hwio.shape
    assert (h2, w2) == (2 * h, 2 * w) and cin_s == cin and cin2 == 2 * cin
    assert kh == 3 and kw == 3
    p = h2 * w2
    cin_p = _round_up(cin, 8)
    c2 = 2 * cin_p
    f32 = jnp.float32
    bf16 = jnp.bfloat16

    aw = _width_matrix(w, w2).astype(bf16)
    h_taps = _height_taps(h, h2)

    xp = x_nchw.astype(f32)
    sp = skip_nchw.astype(f32)
    if cin_p != cin:
        cpad4 = ((0, 0), (0, cin_p - cin), (0, 0), (0, 0))
        xp = jnp.pad(xp, cpad4)
        sp = jnp.pad(sp, cpad4)
    x2d = jnp.transpose(xp, (0, 2, 1, 3)).reshape(n, h * cin_p, w)
    x2d = x2d.astype(bf16)
    s2d = sp.reshape(n, cin_p, p)

    w_up = w_hwio[:, :, :cin, :]
    w_sk = w_hwio[:, :, cin:, :]
    if cin_p != cin:
        wpad = ((0, 0), (0, 0), (0, cin_p - cin), (0, 0))
        w_up = jnp.pad(w_up, wpad)
        w_sk = jnp.pad(w_sk, wpad)
    w_full = jnp.concatenate([w_up, w_sk], axis=2)
    wt = jnp.transpose(w_full, (0, 1, 3, 2))
    ws = jnp.concatenate([wt[:, 1], wt[:, 0], wt[:, 2]], axis=-1)
    ws = ws.reshape(kh * cout, 3 * c2).astype(bf16)

    body = functools.partial(_fwd_kernel, h_taps=h_taps, h2=h2, w2=w2,
                             cin_p=cin_p, cout=cout)

    y_flat, sums, ssqs = pl.pallas_call(
        body,
        out_shape=(jax.ShapeDtypeStruct((n, cout, p), bf16),
                   jax.ShapeDtypeStruct((n, cout, 1), f32),
                   jax.ShapeDtypeStruct((n, cout, 1), f32)),
        grid=(n,),
        in_specs=[
            pl.BlockSpec((w, w2), lambda i: (0, 0)),
            pl.BlockSpec((kh * cout, 3 * c2), lambda i: (0, 0)),
            pl.BlockSpec((1, h * cin_p, w), lambda i: (i, 0, 0)),
            pl.BlockSpec((1, cin_p, p), lambda i: (i, 0, 0)),
        ],
        out_specs=(
            pl.BlockSpec((1, cout, p), lambda i: (i, 0, 0)),
            pl.BlockSpec((1, cout, 1), lambda i: (i, 0, 0)),
            pl.BlockSpec((1, cout, 1), lambda i: (i, 0, 0)),
        ),
        scratch_shapes=[
            pltpu.VMEM((h * cin_p, w2), f32),
            pltpu.VMEM((3 * c2, p), bf16),
            pltpu.VMEM((kh * cout, p), bf16),
        ],
        compiler_params=pltpu.CompilerParams(
            dimension_semantics=("parallel",)),
    )(aw, ws, x2d, s2d)

    cnt = jnp.float32(n * p)
    mean = jnp.sum(sums, axis=0) / cnt
    var = jnp.maximum(jnp.sum(ssqs, axis=0) / cnt - mean * mean, 0.0)
    scale = gamma.reshape(cout, 1).astype(f32) * jax.lax.rsqrt(var + _EPS)
    shift = beta.reshape(cout, 1).astype(f32) - mean * scale

    out_flat = pl.pallas_call(
        _bn_kernel,
        out_shape=jax.ShapeDtypeStruct((n, cout, p), f32),
        grid=(n,),
        in_specs=[
            pl.BlockSpec((cout, 1), lambda i: (0, 0)),
            pl.BlockSpec((cout, 1), lambda i: (0, 0)),
            pl.BlockSpec((1, cout, p), lambda i: (i, 0, 0)),
        ],
        out_specs=pl.BlockSpec((1, cout, p), lambda i: (i, 0, 0)),
        compiler_params=pltpu.CompilerParams(
            dimension_semantics=("parallel",)),
    )(scale, shift, y_flat)

    return out_flat.reshape(n, cout, h2, w2).astype(x_nchw.dtype)
